# baseline (speedup 1.0000x reference)
import functools

import jax
import jax.numpy as jnp
from jax.experimental import pallas as pl
from jax.experimental.pallas import tpu as pltpu

KNN = 20
EPS = 1e-5
RB = 512


def _dot(a, b, dims, precision=jax.lax.Precision.HIGHEST):
    return jax.lax.dot_general(a, b, (dims, ((), ())),
                               preferred_element_type=jnp.float32,
                               precision=precision)


def _k1_body(x_ref, wa_ref, wd_ref, y1_ref, ps_ref):
    rb = pl.program_id(1)
    x = x_ref[0]
    n = x.shape[1]
    G = _dot(wa_ref[...], x, ((1,), (0,)))
    xx = jnp.sum(x * x, axis=0, keepdims=True)
    xr = x_ref[0, :, pl.ds(rb * RB, RB)]
    inner = _dot(xr, x, ((0,), (0,)))
    score = 2.0 * inner - xx
    ccr = _dot(wd_ref[...], xr, ((1,), (0,)))
    iota = jax.lax.broadcasted_iota(jnp.int32, (RB, n), 1)
    ys = []
    s_acc = jnp.zeros((64, 1), jnp.float32)
    q_acc = jnp.zeros((64, 1), jnp.float32)
    for _ in range(KNN):
        mx = jnp.max(score, axis=1, keepdims=True)
        eq = score == mx
        sel = jnp.min(jnp.where(eq, iota, n), axis=1, keepdims=True)
        ohb = iota == sel
        gk = _dot(G, ohb.astype(jnp.float32), ((1,), (1,)))
        yk = gk + ccr
        ys.append(yk)
        s_acc = s_acc + jnp.sum(yk, axis=1, keepdims=True)
        q_acc = q_acc + jnp.sum(yk * yk, axis=1, keepdims=True)
        score = jnp.where(ohb, -jnp.inf, score)
    y1_ref[0] = jnp.stack(ys, axis=1)
    ps_ref[0, 0] = jnp.concatenate([s_acc, q_acc], axis=1)


def _stage_k1(x, wa, wd):
    b, c, n = x.shape
    nb = n // RB
    return pl.pallas_call(
        _k1_body,
        grid=(b, nb),
        in_specs=[
            pl.BlockSpec((1, c, n), lambda i, r: (i, 0, 0)),
            pl.BlockSpec((64, c), lambda i, r: (0, 0)),
            pl.BlockSpec((64, c), lambda i, r: (0, 0)),
        ],
        out_specs=[
            pl.BlockSpec((1, 64, KNN, RB), lambda i, r: (i, 0, 0, r)),
            pl.BlockSpec((1, 1, 64, 2), lambda i, r: (i, r, 0, 0)),
        ],
        out_shape=[
            jax.ShapeDtypeStruct((b, 64, KNN, n), jnp.float32),
            jax.ShapeDtypeStruct((b, nb, 64, 2), jnp.float32),
        ],
        compiler_params=pltpu.CompilerParams(
            dimension_semantics=("parallel", "parallel")),
    )(x, wa, wd)


def _scale_shift(s, g, b, cnt):
    m = s[:, 0:1] / cnt
    v = s[:, 1:2] / cnt - m * m
    inv = jax.lax.rsqrt(v + EPS)
    return inv * g, b - m * inv * g


def _k2_body(y1_ref, ps_ref, g_ref, b_ref, w_ref, y2_ref, p2_ref, *, cnt):
    s = jnp.sum(ps_ref[...], axis=(0, 1))
    scale, shift = _scale_shift(s, g_ref[...], b_ref[...], cnt)
    a = y1_ref[0]
    r = jnp.maximum(a * scale.reshape(64, 1, 1) + shift.reshape(64, 1, 1), 0.0)
    rr = r.reshape(64, KNN * a.shape[2])
    y2 = _dot(w_ref[...], rr, ((1,), (0,)))
    y2_ref[0] = y2.reshape(64, KNN, a.shape[2])
    p2_ref[0] = jnp.concatenate(
        [jnp.sum(y2, axis=1, keepdims=True),
         jnp.sum(y2 * y2, axis=1, keepdims=True)], axis=1)


def _stage_k2(y1, p1, g, bb, w, cnt):
    b = y1.shape[0]
    n = y1.shape[3]
    nb = p1.shape[1]
    return pl.pallas_call(
        functools.partial(_k2_body, cnt=cnt),
        grid=(b,),
        in_specs=[
            pl.BlockSpec((1, 64, KNN, n), lambda i: (i, 0, 0, 0)),
            pl.BlockSpec((b, nb, 64, 2), lambda i: (0, 0, 0, 0)),
            pl.BlockSpec((64, 1), lambda i: (0, 0)),
            pl.BlockSpec((64, 1), lambda i: (0, 0)),
            pl.BlockSpec((64, 64), lambda i: (0, 0)),
        ],
        out_specs=[
            pl.BlockSpec((1, 64, KNN, n), lambda i: (i, 0, 0, 0)),
            pl.BlockSpec((1, 64, 2), lambda i: (i, 0, 0)),
        ],
        out_shape=[
            jax.ShapeDtypeStruct((b, 64, KNN, n), jnp.float32),
            jax.ShapeDtypeStruct((b, 64, 2), jnp.float32),
        ],
        compiler_params=pltpu.CompilerParams(dimension_semantics=("parallel",)),
    )(y1, p1, g, bb, w)


def _k3_body(y2_ref, p2_ref, g_ref, b_ref, o_ref, *, cnt):
    s = jnp.sum(p2_ref[...], axis=0)
    scale, shift = _scale_shift(s, g_ref[...], b_ref[...], cnt)
    a = y2_ref[0]
    r = jnp.maximum(a * scale.reshape(64, 1, 1) + shift.reshape(64, 1, 1), 0.0)
    o_ref[0] = jnp.max(r, axis=1)


def _stage_k3(y2, p2, g, bb, cnt):
    b = y2.shape[0]
    n = y2.shape[3]
    return pl.pallas_call(
        functools.partial(_k3_body, cnt=cnt),
        grid=(b,),
        in_specs=[
            pl.BlockSpec((1, 64, KNN, n), lambda i: (i, 0, 0, 0)),
            pl.BlockSpec((b, 64, 2), lambda i: (0, 0, 0)),
            pl.BlockSpec((64, 1), lambda i: (0, 0)),
            pl.BlockSpec((64, 1), lambda i: (0, 0)),
        ],
        out_specs=pl.BlockSpec((1, 64, n), lambda i: (i, 0, 0)),
        out_shape=jax.ShapeDtypeStruct((b, 64, n), jnp.float32),
        compiler_params=pltpu.CompilerParams(dimension_semantics=("parallel",)),
    )(y2, p2, g, bb)


def _k7_body(c_ref, w_ref, mx_ref, p_ref):
    y = _dot(w_ref[...], c_ref[0], ((1,), (0,)))
    mx_ref[0] = jnp.max(y, axis=1, keepdims=True)
    p_ref[0] = jnp.concatenate(
        [jnp.sum(y, axis=1, keepdims=True),
         jnp.sum(y * y, axis=1, keepdims=True)], axis=1)


def _head_k7(cat, w6):
    b, _, n = cat.shape
    return pl.pallas_call(
        _k7_body,
        grid=(b,),
        in_specs=[
            pl.BlockSpec((1, 192, n), lambda i: (i, 0, 0)),
            pl.BlockSpec((1024, 192), lambda i: (0, 0)),
        ],
        out_specs=[
            pl.BlockSpec((1, 1024, 1), lambda i: (i, 0, 0)),
            pl.BlockSpec((1, 1024, 2), lambda i: (i, 0, 0)),
        ],
        out_shape=[
            jax.ShapeDtypeStruct((b, 1024, 1), jnp.float32),
            jax.ShapeDtypeStruct((b, 1024, 2), jnp.float32),
        ],
        compiler_params=pltpu.CompilerParams(dimension_semantics=("parallel",)),
    )(cat, w6)


def _k8_body(mx_ref, p6_ref, g_ref, b_ref, w7a_ref, w7r_ref, c_ref,
             y7_ref, p7_ref, *, cnt):
    i = pl.program_id(0)
    s = jnp.sum(p6_ref[...], axis=0)
    scale, shift = _scale_shift(s, g_ref[...], b_ref[...], cnt)
    pooled = jnp.maximum(mx_ref[i] * scale + shift, 0.0)
    t = _dot(w7a_ref[...], pooled, ((1,), (0,)))
    y7 = t + _dot(w7r_ref[...], c_ref[0], ((1,), (0,)))
    y7_ref[0] = y7
    p7_ref[0] = jnp.concatenate(
        [jnp.sum(y7, axis=1, keepdims=True),
         jnp.sum(y7 * y7, axis=1, keepdims=True)], axis=1)


def _head_k8(mx6, p6, g, bb, w7a, w7r, cat, cnt):
    b, _, n = cat.shape
    return pl.pallas_call(
        functools.partial(_k8_body, cnt=cnt),
        grid=(b,),
        in_specs=[
            pl.BlockSpec((b, 1024, 1), lambda i: (0, 0, 0)),
            pl.BlockSpec((b, 1024, 2), lambda i: (0, 0, 0)),
            pl.BlockSpec((1024, 1), lambda i: (0, 0)),
            pl.BlockSpec((1024, 1), lambda i: (0, 0)),
            pl.BlockSpec((512, 1024), lambda i: (0, 0)),
            pl.BlockSpec((512, 192), lambda i: (0, 0)),
            pl.BlockSpec((1, 192, n), lambda i: (i, 0, 0)),
        ],
        out_specs=[
            pl.BlockSpec((1, 512, n), lambda i: (i, 0, 0)),
            pl.BlockSpec((1, 512, 2), lambda i: (i, 0, 0)),
        ],
        out_shape=[
            jax.ShapeDtypeStruct((b, 512, n), jnp.float32),
            jax.ShapeDtypeStruct((b, 512, 2), jnp.float32),
        ],
        compiler_params=pltpu.CompilerParams(dimension_semantics=("parallel",)),
    )(mx6, p6, g, bb, w7a, w7r, cat)


def _k9_body(y7_ref, p7_ref, g_ref, b_ref, w_ref, y8_ref, p8_ref, *, cnt, co):
    s = jnp.sum(p7_ref[...], axis=0)
    scale, shift = _scale_shift(s, g_ref[...], b_ref[...], cnt)
    r = jnp.maximum(y7_ref[0] * scale + shift, 0.0)
    y8 = _dot(w_ref[...], r, ((1,), (0,)))
    y8_ref[0] = y8
    p8_ref[0] = jnp.concatenate(
        [jnp.sum(y8, axis=1, keepdims=True),
         jnp.sum(y8 * y8, axis=1, keepdims=True)], axis=1)


def _head_k9(y7, p7, g, bb, w8, cnt):
    b, ci, n = y7.shape
    co = w8.shape[0]
    return pl.pallas_call(
        functools.partial(_k9_body, cnt=cnt, co=co),
        grid=(b,),
        in_specs=[
            pl.BlockSpec((1, ci, n), lambda i: (i, 0, 0)),
            pl.BlockSpec((b, ci, 2), lambda i: (0, 0, 0)),
            pl.BlockSpec((ci, 1), lambda i: (0, 0)),
            pl.BlockSpec((ci, 1), lambda i: (0, 0)),
            pl.BlockSpec((co, ci), lambda i: (0, 0)),
        ],
        out_specs=[
            pl.BlockSpec((1, co, n), lambda i: (i, 0, 0)),
            pl.BlockSpec((1, co, 2), lambda i: (i, 0, 0)),
        ],
        out_shape=[
            jax.ShapeDtypeStruct((b, co, n), jnp.float32),
            jax.ShapeDtypeStruct((b, co, 2), jnp.float32),
        ],
        compiler_params=pltpu.CompilerParams(dimension_semantics=("parallel",)),
    )(y7, p7, g, bb, w8)


def _k10_body(y8_ref, p8_ref, g_ref, b_ref, w_ref, o_ref, *, cnt):
    s = jnp.sum(p8_ref[...], axis=0)
    scale, shift = _scale_shift(s, g_ref[...], b_ref[...], cnt)
    r = jnp.maximum(y8_ref[0] * scale + shift, 0.0)
    o_ref[0] = _dot(w_ref[...], r, ((1,), (0,)))


def _head_k10(y8, p8, g, bb, w9, cnt):
    b, ci, n = y8.shape
    return pl.pallas_call(
        functools.partial(_k10_body, cnt=cnt),
        grid=(b,),
        in_specs=[
            pl.BlockSpec((1, ci, n), lambda i: (i, 0, 0)),
            pl.BlockSpec((b, ci, 2), lambda i: (0, 0, 0)),
            pl.BlockSpec((ci, 1), lambda i: (0, 0)),
            pl.BlockSpec((ci, 1), lambda i: (0, 0)),
            pl.BlockSpec((1, ci), lambda i: (0, 0)),
        ],
        out_specs=pl.BlockSpec((1, 1, n), lambda i: (i, 0, 0)),
        out_shape=jax.ShapeDtypeStruct((b, 1, n), jnp.float32),
        compiler_params=pltpu.CompilerParams(dimension_semantics=("parallel",)),
    )(y8, p8, g, bb, w9)


def kernel(x, Ws, gs, bs):
    b, c0, n = x.shape
    gcol = [g.reshape(-1, 1) for g in gs]
    bcol = [v.reshape(-1, 1) for v in bs]
    cnt_k = b * n * KNN
    cnt_n = b * n

    def stage(xin, w1st, g1, b1, w2nd, g2, b2):
        c = xin.shape[1]
        wa, wb = w1st[:, :c], w1st[:, c:]
        y1, p1 = _stage_k1(xin, wa, wb - wa)
        y2, p2 = _stage_k2(y1, p1, g1, b1, w2nd, cnt_k)
        return _stage_k3(y2, p2, g2, b2, cnt_k)

    x1 = stage(x, Ws[0], gcol[0], bcol[0], Ws[1], gcol[1], bcol[1])
    x2 = stage(x1, Ws[2], gcol[2], bcol[2], Ws[3], gcol[3], bcol[3])
    x3 = stage(x2, Ws[4], gcol[4], bcol[4], Ws[5], gcol[5], bcol[5])
    cat = jnp.concatenate([x1, x2, x3], axis=1)
    mx6, p6 = _head_k7(cat, Ws[6])
    y7, p7 = _head_k8(mx6, p6, gcol[6], bcol[6],
                      Ws[7][:, :1024], Ws[7][:, 1024:], cat, cnt_n)
    y8, p8 = _head_k9(y7, p7, gcol[7], bcol[7], Ws[8], cnt_n)
    return _head_k10(y8, p8, gcol[8], bcol[8], Ws[9], cnt_n).reshape(b, n)


# speedup vs baseline: 1.2818x; 1.2818x over previous
import functools

import jax
import jax.numpy as jnp
from jax.experimental import pallas as pl
from jax.experimental.pallas import tpu as pltpu

KNN = 20
EPS = 1e-5
RB = 128


def _dot(a, b, dims, precision=jax.lax.Precision.HIGHEST):
    return jax.lax.dot_general(a, b, (dims, ((), ())),
                               preferred_element_type=jnp.float32,
                               precision=precision)


def _k1_body(x_ref, wa_ref, wd_ref, y1_ref, ps_ref):
    rb = pl.program_id(1)
    x = x_ref[0]
    n = x.shape[1]
    G = _dot(wa_ref[...], x, ((1,), (0,)))
    xx = jnp.sum(x * x, axis=0, keepdims=True)
    xr = x_ref[0, :, pl.ds(rb * RB, RB)]
    inner = _dot(xr, x, ((0,), (0,)))
    score = 2.0 * inner - xx
    ccr = _dot(wd_ref[...], xr, ((1,), (0,)))
    iota = jax.lax.broadcasted_iota(jnp.int32, (RB, n), 1)
    ys = []
    s_acc = jnp.zeros((64, 1), jnp.float32)
    q_acc = jnp.zeros((64, 1), jnp.float32)
    for _ in range(KNN):
        mx = jnp.max(score, axis=1, keepdims=True)
        eq = score == mx
        sel = jnp.min(jnp.where(eq, iota, n), axis=1, keepdims=True)
        ohb = iota == sel
        gk = _dot(G, ohb.astype(jnp.float32), ((1,), (1,)))
        yk = gk + ccr
        ys.append(yk)
        s_acc = s_acc + jnp.sum(yk, axis=1, keepdims=True)
        q_acc = q_acc + jnp.sum(yk * yk, axis=1, keepdims=True)
        score = jnp.where(ohb, -jnp.inf, score)
    y1_ref[0] = jnp.stack(ys, axis=1)
    ps_ref[0, 0] = jnp.concatenate([s_acc, q_acc], axis=1)


def _stage_k1(x, wa, wd):
    b, c, n = x.shape
    nb = n // RB
    return pl.pallas_call(
        _k1_body,
        grid=(b, nb),
        in_specs=[
            pl.BlockSpec((1, c, n), lambda i, r: (i, 0, 0)),
            pl.BlockSpec((64, c), lambda i, r: (0, 0)),
            pl.BlockSpec((64, c), lambda i, r: (0, 0)),
        ],
        out_specs=[
            pl.BlockSpec((1, 64, KNN, RB), lambda i, r: (i, 0, 0, r)),
            pl.BlockSpec((1, 1, 64, 2), lambda i, r: (i, r, 0, 0)),
        ],
        out_shape=[
            jax.ShapeDtypeStruct((b, 64, KNN, n), jnp.float32),
            jax.ShapeDtypeStruct((b, nb, 64, 2), jnp.float32),
        ],
        compiler_params=pltpu.CompilerParams(
            dimension_semantics=("parallel", "parallel")),
    )(x, wa, wd)


def _scale_shift(s, g, b, cnt):
    m = s[:, 0:1] / cnt
    v = s[:, 1:2] / cnt - m * m
    inv = jax.lax.rsqrt(v + EPS)
    return inv * g, b - m * inv * g


def _k2_body(y1_ref, ps_ref, g_ref, b_ref, w_ref, y2_ref, p2_ref, *, cnt):
    s = jnp.sum(ps_ref[...], axis=(0, 1))
    scale, shift = _scale_shift(s, g_ref[...], b_ref[...], cnt)
    a = y1_ref[0]
    r = jnp.maximum(a * scale.reshape(64, 1, 1) + shift.reshape(64, 1, 1), 0.0)
    rr = r.reshape(64, KNN * a.shape[2])
    y2 = _dot(w_ref[...], rr, ((1,), (0,)))
    y2_ref[0] = y2.reshape(64, KNN, a.shape[2])
    p2_ref[0] = jnp.concatenate(
        [jnp.sum(y2, axis=1, keepdims=True),
         jnp.sum(y2 * y2, axis=1, keepdims=True)], axis=1)


def _stage_k2(y1, p1, g, bb, w, cnt):
    b = y1.shape[0]
    n = y1.shape[3]
    nb = p1.shape[1]
    return pl.pallas_call(
        functools.partial(_k2_body, cnt=cnt),
        grid=(b,),
        in_specs=[
            pl.BlockSpec((1, 64, KNN, n), lambda i: (i, 0, 0, 0)),
            pl.BlockSpec((b, nb, 64, 2), lambda i: (0, 0, 0, 0)),
            pl.BlockSpec((64, 1), lambda i: (0, 0)),
            pl.BlockSpec((64, 1), lambda i: (0, 0)),
            pl.BlockSpec((64, 64), lambda i: (0, 0)),
        ],
        out_specs=[
            pl.BlockSpec((1, 64, KNN, n), lambda i: (i, 0, 0, 0)),
            pl.BlockSpec((1, 64, 2), lambda i: (i, 0, 0)),
        ],
        out_shape=[
            jax.ShapeDtypeStruct((b, 64, KNN, n), jnp.float32),
            jax.ShapeDtypeStruct((b, 64, 2), jnp.float32),
        ],
        compiler_params=pltpu.CompilerParams(dimension_semantics=("parallel",)),
    )(y1, p1, g, bb, w)


def _k3_body(y2_ref, p2_ref, g_ref, b_ref, o_ref, *, cnt):
    s = jnp.sum(p2_ref[...], axis=0)
    scale, shift = _scale_shift(s, g_ref[...], b_ref[...], cnt)
    a = y2_ref[0]
    r = jnp.maximum(a * scale.reshape(64, 1, 1) + shift.reshape(64, 1, 1), 0.0)
    o_ref[0] = jnp.max(r, axis=1)


def _stage_k3(y2, p2, g, bb, cnt):
    b = y2.shape[0]
    n = y2.shape[3]
    return pl.pallas_call(
        functools.partial(_k3_body, cnt=cnt),
        grid=(b,),
        in_specs=[
            pl.BlockSpec((1, 64, KNN, n), lambda i: (i, 0, 0, 0)),
            pl.BlockSpec((b, 64, 2), lambda i: (0, 0, 0)),
            pl.BlockSpec((64, 1), lambda i: (0, 0)),
            pl.BlockSpec((64, 1), lambda i: (0, 0)),
        ],
        out_specs=pl.BlockSpec((1, 64, n), lambda i: (i, 0, 0)),
        out_shape=jax.ShapeDtypeStruct((b, 64, n), jnp.float32),
        compiler_params=pltpu.CompilerParams(dimension_semantics=("parallel",)),
    )(y2, p2, g, bb)


def _k7_body(c_ref, w_ref, mx_ref, p_ref):
    y = _dot(w_ref[...], c_ref[0], ((1,), (0,)))
    mx_ref[0] = jnp.max(y, axis=1, keepdims=True)
    p_ref[0] = jnp.concatenate(
        [jnp.sum(y, axis=1, keepdims=True),
         jnp.sum(y * y, axis=1, keepdims=True)], axis=1)


def _head_k7(cat, w6):
    b, _, n = cat.shape
    return pl.pallas_call(
        _k7_body,
        grid=(b,),
        in_specs=[
            pl.BlockSpec((1, 192, n), lambda i: (i, 0, 0)),
            pl.BlockSpec((1024, 192), lambda i: (0, 0)),
        ],
        out_specs=[
            pl.BlockSpec((1, 1024, 1), lambda i: (i, 0, 0)),
            pl.BlockSpec((1, 1024, 2), lambda i: (i, 0, 0)),
        ],
        out_shape=[
            jax.ShapeDtypeStruct((b, 1024, 1), jnp.float32),
            jax.ShapeDtypeStruct((b, 1024, 2), jnp.float32),
        ],
        compiler_params=pltpu.CompilerParams(dimension_semantics=("parallel",)),
    )(cat, w6)


def _k8_body(mx_ref, p6_ref, g_ref, b_ref, w7a_ref, w7r_ref, c_ref,
             y7_ref, p7_ref, *, cnt):
    i = pl.program_id(0)
    s = jnp.sum(p6_ref[...], axis=0)
    scale, shift = _scale_shift(s, g_ref[...], b_ref[...], cnt)
    pooled = jnp.maximum(mx_ref[i] * scale + shift, 0.0)
    t = _dot(w7a_ref[...], pooled, ((1,), (0,)))
    y7 = t + _dot(w7r_ref[...], c_ref[0], ((1,), (0,)))
    y7_ref[0] = y7
    p7_ref[0] = jnp.concatenate(
        [jnp.sum(y7, axis=1, keepdims=True),
         jnp.sum(y7 * y7, axis=1, keepdims=True)], axis=1)


def _head_k8(mx6, p6, g, bb, w7a, w7r, cat, cnt):
    b, _, n = cat.shape
    return pl.pallas_call(
        functools.partial(_k8_body, cnt=cnt),
        grid=(b,),
        in_specs=[
            pl.BlockSpec((b, 1024, 1), lambda i: (0, 0, 0)),
            pl.BlockSpec((b, 1024, 2), lambda i: (0, 0, 0)),
            pl.BlockSpec((1024, 1), lambda i: (0, 0)),
            pl.BlockSpec((1024, 1), lambda i: (0, 0)),
            pl.BlockSpec((512, 1024), lambda i: (0, 0)),
            pl.BlockSpec((512, 192), lambda i: (0, 0)),
            pl.BlockSpec((1, 192, n), lambda i: (i, 0, 0)),
        ],
        out_specs=[
            pl.BlockSpec((1, 512, n), lambda i: (i, 0, 0)),
            pl.BlockSpec((1, 512, 2), lambda i: (i, 0, 0)),
        ],
        out_shape=[
            jax.ShapeDtypeStruct((b, 512, n), jnp.float32),
            jax.ShapeDtypeStruct((b, 512, 2), jnp.float32),
        ],
        compiler_params=pltpu.CompilerParams(dimension_semantics=("parallel",)),
    )(mx6, p6, g, bb, w7a, w7r, cat)


def _k9_body(y7_ref, p7_ref, g_ref, b_ref, w_ref, y8_ref, p8_ref, *, cnt, co):
    s = jnp.sum(p7_ref[...], axis=0)
    scale, shift = _scale_shift(s, g_ref[...], b_ref[...], cnt)
    r = jnp.maximum(y7_ref[0] * scale + shift, 0.0)
    y8 = _dot(w_ref[...], r, ((1,), (0,)))
    y8_ref[0] = y8
    p8_ref[0] = jnp.concatenate(
        [jnp.sum(y8, axis=1, keepdims=True),
         jnp.sum(y8 * y8, axis=1, keepdims=True)], axis=1)


def _head_k9(y7, p7, g, bb, w8, cnt):
    b, ci, n = y7.shape
    co = w8.shape[0]
    return pl.pallas_call(
        functools.partial(_k9_body, cnt=cnt, co=co),
        grid=(b,),
        in_specs=[
            pl.BlockSpec((1, ci, n), lambda i: (i, 0, 0)),
            pl.BlockSpec((b, ci, 2), lambda i: (0, 0, 0)),
            pl.BlockSpec((ci, 1), lambda i: (0, 0)),
            pl.BlockSpec((ci, 1), lambda i: (0, 0)),
            pl.BlockSpec((co, ci), lambda i: (0, 0)),
        ],
        out_specs=[
            pl.BlockSpec((1, co, n), lambda i: (i, 0, 0)),
            pl.BlockSpec((1, co, 2), lambda i: (i, 0, 0)),
        ],
        out_shape=[
            jax.ShapeDtypeStruct((b, co, n), jnp.float32),
            jax.ShapeDtypeStruct((b, co, 2), jnp.float32),
        ],
        compiler_params=pltpu.CompilerParams(dimension_semantics=("parallel",)),
    )(y7, p7, g, bb, w8)


def _k10_body(y8_ref, p8_ref, g_ref, b_ref, w_ref, o_ref, *, cnt):
    s = jnp.sum(p8_ref[...], axis=0)
    scale, shift = _scale_shift(s, g_ref[...], b_ref[...], cnt)
    r = jnp.maximum(y8_ref[0] * scale + shift, 0.0)
    o_ref[0] = _dot(w_ref[...], r, ((1,), (0,)))


def _head_k10(y8, p8, g, bb, w9, cnt):
    b, ci, n = y8.shape
    return pl.pallas_call(
        functools.partial(_k10_body, cnt=cnt),
        grid=(b,),
        in_specs=[
            pl.BlockSpec((1, ci, n), lambda i: (i, 0, 0)),
            pl.BlockSpec((b, ci, 2), lambda i: (0, 0, 0)),
            pl.BlockSpec((ci, 1), lambda i: (0, 0)),
            pl.BlockSpec((ci, 1), lambda i: (0, 0)),
            pl.BlockSpec((1, ci), lambda i: (0, 0)),
        ],
        out_specs=pl.BlockSpec((1, 1, n), lambda i: (i, 0, 0)),
        out_shape=jax.ShapeDtypeStruct((b, 1, n), jnp.float32),
        compiler_params=pltpu.CompilerParams(dimension_semantics=("parallel",)),
    )(y8, p8, g, bb, w9)


def kernel(x, Ws, gs, bs):
    b, c0, n = x.shape
    gcol = [g.reshape(-1, 1) for g in gs]
    bcol = [v.reshape(-1, 1) for v in bs]
    cnt_k = b * n * KNN
    cnt_n = b * n

    def stage(xin, w1st, g1, b1, w2nd, g2, b2):
        c = xin.shape[1]
        wa, wb = w1st[:, :c], w1st[:, c:]
        y1, p1 = _stage_k1(xin, wa, wb - wa)
        y2, p2 = _stage_k2(y1, p1, g1, b1, w2nd, cnt_k)
        return _stage_k3(y2, p2, g2, b2, cnt_k)

    x1 = stage(x, Ws[0], gcol[0], bcol[0], Ws[1], gcol[1], bcol[1])
    x2 = stage(x1, Ws[2], gcol[2], bcol[2], Ws[3], gcol[3], bcol[3])
    x3 = stage(x2, Ws[4], gcol[4], bcol[4], Ws[5], gcol[5], bcol[5])
    cat = jnp.concatenate([x1, x2, x3], axis=1)
    mx6, p6 = _head_k7(cat, Ws[6])
    y7, p7 = _head_k8(mx6, p6, gcol[6], bcol[6],
                      Ws[7][:, :1024], Ws[7][:, 1024:], cat, cnt_n)
    y8, p8 = _head_k9(y7, p7, gcol[7], bcol[7], Ws[8], cnt_n)
    return _head_k10(y8, p8, gcol[8], bcol[8], Ws[9], cnt_n).reshape(b, n)
